# baseline (speedup 1.0000x reference)
import jax
import jax.numpy as jnp
from jax import lax
from jax.experimental import pallas as pl

PENALTY = 0.1
NEG_SLOPE = 0.01


def _lrelu(v):
    return jnp.where(v >= 0, v, NEG_SLOPE * v)


def _dot_hi(a, b):
    return jnp.dot(a, b, preferred_element_type=jnp.float32,
                   precision=lax.Precision.HIGHEST)


def _stnet_body(x_ref, src_ref, dst_ref, Wg_ref, bg_ref, Wggc_ref, Wih_ref,
                Whh_ref, bih_ref, bhh_ref, W1_ref, b1_ref, W2_ref, b2_ref,
                s_ref, min_ref, loss_ref):
    f32 = jnp.float32
    n = x_ref.shape[0]
    e = src_ref.shape[0]
    eb = 2048
    num_l = Wggc_ref.shape[0]
    h_dim = Wg_ref.shape[1]

    iota_col = lax.broadcasted_iota(jnp.int32, (n, 1), 0)
    iota_row = lax.broadcasted_iota(jnp.int32, (1, n), 1)

    Ai = jnp.zeros((n, n), f32)
    for b in range(e // eb):
        sblk = src_ref[pl.ds(b * eb, eb), :]
        dblk = dst_ref[:, pl.ds(b * eb, eb)]
        iota_e = lax.broadcasted_iota(jnp.int32, (eb, n), 1)
        Os = (sblk == iota_e).astype(jnp.bfloat16)
        OdT = (iota_col == dblk).astype(jnp.bfloat16)
        Ai = Ai + jnp.dot(OdT, Os, preferred_element_type=f32)
    A = Ai

    rowsum = jnp.sum(A, axis=1, keepdims=True)
    deg = jnp.maximum(rowsum, 1.0)
    r = lax.rsqrt(deg)

    xw = _dot_hi(x_ref[...], Wg_ref[...])
    agg = r * _dot_hi(A, r * xw) + bg_ref[...]
    x1 = _lrelu(agg)

    h = x1
    for i in range(num_l):
        hw = _dot_hi(h, Wggc_ref[i])
        m = _dot_hi(A, hw)
        gi = _dot_hi(m, Wih_ref[...]) + bih_ref[...]
        gh = _dot_hi(h, Whh_ref[...]) + bhh_ref[...]
        rg = jax.nn.sigmoid(gi[:, :h_dim] + gh[:, :h_dim])
        z = jax.nn.sigmoid(gi[:, h_dim:2 * h_dim] + gh[:, h_dim:2 * h_dim])
        nn_ = jnp.tanh(gi[:, 2 * h_dim:] + rg * gh[:, 2 * h_dim:])
        h = (1.0 - z) * nn_ + z * h

    x2 = _lrelu(h) + x1
    x3 = _lrelu(_dot_hi(x2, W1_ref[...]) + b1_ref[...])
    xf_col = jax.nn.sigmoid(_lrelu(_dot_hi(x3, W2_ref[...]) + b2_ref[...]))

    eye = (iota_col == iota_row).astype(f32)
    xf_row = lax.dot_general(xf_col, eye, (((0,), (0,)), ((), ())),
                             preferred_element_type=f32,
                             precision=lax.Precision.HIGHEST)

    gt = (xf_col > xf_row).astype(f32)
    tie = jnp.logical_and(xf_col == xf_row, iota_col < iota_row).astype(f32)
    rank_row = jnp.sum(gt + tie, axis=0, keepdims=True)

    icolf = iota_col.astype(f32)
    M = (rank_row <= icolf).astype(f32)
    colsum = lax.dot_general(A, jnp.ones((n, 1), f32), (((0,), (0,)), ((), ())),
                             preferred_element_type=f32)
    rc = rowsum + colsum
    t12 = jnp.dot(M, rc, preferred_element_type=f32)
    Bm = jnp.dot(M, A, preferred_element_type=f32)
    t3 = jnp.sum(Bm * M, axis=1, keepdims=True)
    cut = t12 - 2.0 * t3
    f_unreg = -cut
    f_sets = f_unreg + PENALTY * (icolf + 1.0)

    s_ref[...] = xf_col
    min_ref[...] = jnp.min(f_unreg, axis=0, keepdims=True)
    loss_ref[...] = jnp.sum(f_sets, axis=0, keepdims=True) / n


def kernel(x, edge_index, batch, W_gcn, b_gcn, W_ggc, W_ih, W_hh, b_ih, b_hh,
           W_lin1, b_lin1, W_lin2, b_lin2):
    n = x.shape[0]
    e = edge_index.shape[1]
    f32 = jnp.float32
    src_col = edge_index[0].reshape(e, 1)
    dst_row = edge_index[1].reshape(1, e)
    s, mn, ls = pl.pallas_call(
        _stnet_body,
        out_shape=(
            jax.ShapeDtypeStruct((n, 1), f32),
            jax.ShapeDtypeStruct((1, 1), f32),
            jax.ShapeDtypeStruct((1, 1), f32),
        ),
    )(x, src_col, dst_row, W_gcn, b_gcn, W_ggc, W_ih, W_hh, b_ih, b_hh,
      W_lin1, b_lin1, W_lin2, b_lin2)
    return (s, mn.reshape(()), ls.reshape(()))


# speedup vs baseline: 1.2856x; 1.2856x over previous
import jax
import jax.numpy as jnp
from jax import lax
from jax.experimental import pallas as pl

PENALTY = 0.1
NEG_SLOPE = 0.01


def _lrelu(v):
    return jnp.where(v >= 0, v, NEG_SLOPE * v)


def _dot_hi(a, b):
    return jnp.dot(a, b, preferred_element_type=jnp.float32,
                   precision=lax.Precision.HIGHEST)


def _stnet_body(x_ref, src_ref, dst_ref, Wg_ref, bg_ref, Wggc_ref, Wih_ref,
                Whh_ref, bih_ref, bhh_ref, W1_ref, b1_ref, W2_ref, b2_ref,
                s_ref, min_ref, loss_ref):
    f32 = jnp.float32
    n = x_ref.shape[0]
    e = src_ref.shape[0]
    eb = 2048
    num_l = Wggc_ref.shape[0]
    h_dim = Wg_ref.shape[1]

    iota_col = lax.broadcasted_iota(jnp.int32, (n, 1), 0)
    iota_row = lax.broadcasted_iota(jnp.int32, (1, n), 1)

    Ai = jnp.zeros((n, n), f32)
    for b in range(e // eb):
        sblk = src_ref[pl.ds(b * eb, eb), :]
        dblk = dst_ref[:, pl.ds(b * eb, eb)]
        iota_e = lax.broadcasted_iota(jnp.int32, (eb, n), 1)
        Os = (sblk == iota_e).astype(jnp.bfloat16)
        OdT = (iota_col == dblk).astype(jnp.bfloat16)
        Ai = Ai + jnp.dot(OdT, Os, preferred_element_type=f32)
    A = Ai

    rowsum = jnp.sum(A, axis=1, keepdims=True)
    deg = jnp.maximum(rowsum, 1.0)

    eye = (iota_col == iota_row).astype(f32)
    deg_row = lax.dot_general(deg, eye, (((0,), (0,)), ((), ())),
                              preferred_element_type=f32,
                              precision=lax.Precision.HIGHEST)
    An = A * lax.rsqrt(deg * deg_row)
    xw = jnp.dot(x_ref[...], Wg_ref[...], preferred_element_type=f32)
    agg = _dot_hi(An, xw) + bg_ref[...]
    x1 = _lrelu(agg)

    h = x1
    for i in range(num_l):
        hw = jnp.dot(h, Wggc_ref[i], preferred_element_type=f32)
        m = _dot_hi(A, hw)
        gi = jnp.dot(m, Wih_ref[...], preferred_element_type=f32) + bih_ref[...]
        gh = jnp.dot(h, Whh_ref[...], preferred_element_type=f32) + bhh_ref[...]
        rg = jax.nn.sigmoid(gi[:, :h_dim] + gh[:, :h_dim])
        z = jax.nn.sigmoid(gi[:, h_dim:2 * h_dim] + gh[:, h_dim:2 * h_dim])
        nn_ = jnp.tanh(gi[:, 2 * h_dim:] + rg * gh[:, 2 * h_dim:])
        h = (1.0 - z) * nn_ + z * h

    x2 = _lrelu(h) + x1
    x3 = _lrelu(jnp.dot(x2, W1_ref[...], preferred_element_type=f32) + b1_ref[...])
    xf_col = jax.nn.sigmoid(_lrelu(jnp.dot(x3, W2_ref[...],
                                           preferred_element_type=f32) + b2_ref[...]))

    xf_row = lax.dot_general(xf_col, eye, (((0,), (0,)), ((), ())),
                             preferred_element_type=f32,
                             precision=lax.Precision.HIGHEST)

    gt = (xf_col > xf_row).astype(f32)
    tie = jnp.logical_and(xf_col == xf_row, iota_col < iota_row).astype(f32)
    rank_row = jnp.sum(gt + tie, axis=0, keepdims=True)

    icolf = iota_col.astype(f32)
    M = (rank_row <= icolf).astype(f32)
    colsum = lax.dot_general(A, jnp.ones((n, 1), f32), (((0,), (0,)), ((), ())),
                             preferred_element_type=f32)
    rc = rowsum + colsum
    t12 = jnp.dot(M, rc, preferred_element_type=f32)
    Bm = jnp.dot(M, A, preferred_element_type=f32)
    t3 = jnp.sum(Bm * M, axis=1, keepdims=True)
    cut = t12 - 2.0 * t3
    f_unreg = -cut
    f_sets = f_unreg + PENALTY * (icolf + 1.0)

    s_ref[...] = xf_col
    min_ref[...] = jnp.min(f_unreg, axis=0, keepdims=True)
    loss_ref[...] = jnp.sum(f_sets, axis=0, keepdims=True) / n


def kernel(x, edge_index, batch, W_gcn, b_gcn, W_ggc, W_ih, W_hh, b_ih, b_hh,
           W_lin1, b_lin1, W_lin2, b_lin2):
    n = x.shape[0]
    e = edge_index.shape[1]
    f32 = jnp.float32
    src_col = edge_index[0].reshape(e, 1)
    dst_row = edge_index[1].reshape(1, e)
    s, mn, ls = pl.pallas_call(
        _stnet_body,
        out_shape=(
            jax.ShapeDtypeStruct((n, 1), f32),
            jax.ShapeDtypeStruct((1, 1), f32),
            jax.ShapeDtypeStruct((1, 1), f32),
        ),
    )(x, src_col, dst_row, W_gcn, b_gcn, W_ggc, W_ih, W_hh, b_ih, b_hh,
      W_lin1, b_lin1, W_lin2, b_lin2)
    return (s, mn.reshape(()), ls.reshape(()))


# speedup vs baseline: 1.5178x; 1.1806x over previous
import jax
import jax.numpy as jnp
from jax import lax
from jax.experimental import pallas as pl

PENALTY = 0.1
NEG_SLOPE = 0.01


def _lrelu(v):
    return jnp.where(v >= 0, v, NEG_SLOPE * v)


def _dot_hi(a, b):
    return jnp.dot(a, b, preferred_element_type=jnp.float32,
                   precision=lax.Precision.HIGHEST)


def _seg_dot(a16, v):
    f32 = jnp.float32
    hi = v.astype(jnp.bfloat16)
    r1 = v - hi.astype(f32)
    mid = r1.astype(jnp.bfloat16)
    lo = (r1 - mid.astype(f32)).astype(jnp.bfloat16)
    acc = jnp.dot(a16, hi, preferred_element_type=f32)
    acc = acc + jnp.dot(a16, mid, preferred_element_type=f32)
    acc = acc + jnp.dot(a16, lo, preferred_element_type=f32)
    return acc


def _stnet_body(x_ref, src_ref, dst_ref, Wg_ref, bg_ref, Wggc_ref, Wih_ref,
                Whh_ref, bih_ref, bhh_ref, W1_ref, b1_ref, W2_ref, b2_ref,
                s_ref, min_ref, loss_ref):
    f32 = jnp.float32
    n = x_ref.shape[0]
    e = src_ref.shape[0]
    eb = 2048
    num_l = Wggc_ref.shape[0]
    h_dim = Wg_ref.shape[1]

    iota_col = lax.broadcasted_iota(jnp.int32, (n, 1), 0)
    iota_row = lax.broadcasted_iota(jnp.int32, (1, n), 1)

    Ai = jnp.zeros((n, n), f32)
    for b in range(e // eb):
        sblk = src_ref[pl.ds(b * eb, eb), :]
        dblk = dst_ref[:, pl.ds(b * eb, eb)]
        iota_e = lax.broadcasted_iota(jnp.int32, (eb, n), 1)
        Os = (sblk == iota_e).astype(jnp.bfloat16)
        OdT = (iota_col == dblk).astype(jnp.bfloat16)
        Ai = Ai + jnp.dot(OdT, Os, preferred_element_type=f32)
    A = Ai

    rowsum = jnp.sum(A, axis=1, keepdims=True)
    deg = jnp.maximum(rowsum, 1.0)
    r = lax.rsqrt(deg)
    A16 = A.astype(jnp.bfloat16)

    xw = jnp.dot(x_ref[...], Wg_ref[...], preferred_element_type=f32)
    agg = r * _seg_dot(A16, r * xw) + bg_ref[...]
    x1 = _lrelu(agg)

    h = x1
    for i in range(num_l):
        hw = jnp.dot(h, Wggc_ref[i], preferred_element_type=f32)
        m = _seg_dot(A16, hw)
        gi = jnp.dot(m, Wih_ref[...], preferred_element_type=f32) + bih_ref[...]
        gh = jnp.dot(h, Whh_ref[...], preferred_element_type=f32) + bhh_ref[...]
        rg = jax.nn.sigmoid(gi[:, :h_dim] + gh[:, :h_dim])
        z = jax.nn.sigmoid(gi[:, h_dim:2 * h_dim] + gh[:, h_dim:2 * h_dim])
        nn_ = jnp.tanh(gi[:, 2 * h_dim:] + rg * gh[:, 2 * h_dim:])
        h = (1.0 - z) * nn_ + z * h

    x2 = _lrelu(h) + x1
    x3 = _lrelu(jnp.dot(x2, W1_ref[...], preferred_element_type=f32) + b1_ref[...])
    xf_col = jax.nn.sigmoid(_lrelu(jnp.dot(x3, W2_ref[...],
                                           preferred_element_type=f32) + b2_ref[...]))

    eye = (iota_col == iota_row).astype(f32)
    xf_row = lax.dot_general(xf_col, eye, (((0,), (0,)), ((), ())),
                             preferred_element_type=f32,
                             precision=lax.Precision.HIGHEST)

    gt = (xf_col > xf_row).astype(f32)
    tie = jnp.logical_and(xf_col == xf_row, iota_col < iota_row).astype(f32)
    rank_row = jnp.sum(gt + tie, axis=0, keepdims=True)

    icolf = iota_col.astype(f32)
    M = (rank_row <= icolf).astype(f32)
    colsum = lax.dot_general(A, jnp.ones((n, 1), f32), (((0,), (0,)), ((), ())),
                             preferred_element_type=f32)
    rc = rowsum + colsum
    t12 = jnp.dot(M, rc, preferred_element_type=f32)
    Bm = jnp.dot(M, A, preferred_element_type=f32)
    t3 = jnp.sum(Bm * M, axis=1, keepdims=True)
    cut = t12 - 2.0 * t3
    f_unreg = -cut
    f_sets = f_unreg + PENALTY * (icolf + 1.0)

    s_ref[...] = xf_col
    min_ref[...] = jnp.min(f_unreg, axis=0, keepdims=True)
    loss_ref[...] = jnp.sum(f_sets, axis=0, keepdims=True) / n


def kernel(x, edge_index, batch, W_gcn, b_gcn, W_ggc, W_ih, W_hh, b_ih, b_hh,
           W_lin1, b_lin1, W_lin2, b_lin2):
    n = x.shape[0]
    e = edge_index.shape[1]
    f32 = jnp.float32
    src_col = edge_index[0].reshape(e, 1)
    dst_row = edge_index[1].reshape(1, e)
    s, mn, ls = pl.pallas_call(
        _stnet_body,
        out_shape=(
            jax.ShapeDtypeStruct((n, 1), f32),
            jax.ShapeDtypeStruct((1, 1), f32),
            jax.ShapeDtypeStruct((1, 1), f32),
        ),
    )(x, src_col, dst_row, W_gcn, b_gcn, W_ggc, W_ih, W_hh, b_ih, b_hh,
      W_lin1, b_lin1, W_lin2, b_lin2)
    return (s, mn.reshape(()), ls.reshape(()))


# speedup vs baseline: 1.5215x; 1.0025x over previous
import functools

import jax
import jax.numpy as jnp
from jax import lax
from jax.experimental import pallas as pl
from jax.experimental.pallas import tpu as pltpu, tpu_sc as plsc

PENALTY = 0.1
NEG_SLOPE = 0.01
_N = 1024
_E = 16384
_NW = 32
_RPW = _N // _NW


_sc_mesh = plsc.VectorSubcoreMesh(core_axis_name="c", subcore_axis_name="s")


@functools.partial(
    pl.kernel, mesh=_sc_mesh,
    out_type=jax.ShapeDtypeStruct((_N, _N), jnp.float32),
    compiler_params=pltpu.CompilerParams(needs_layout_passes=False),
    scratch_types=[
        pltpu.VMEM((_RPW, _N), jnp.float32),
        pltpu.VMEM((_E,), jnp.int32),
        pltpu.VMEM((_E,), jnp.int32),
    ],
)
def _build_adjacency(src_hbm, dst_hbm, a_hbm, a_v, src_v, dst_v):
    wid = lax.axis_index("s") * 2 + lax.axis_index("c")
    base = wid * _RPW
    pltpu.sync_copy(src_hbm, src_v)
    pltpu.sync_copy(dst_hbm, dst_v)

    zeros16 = jnp.zeros((16,), jnp.float32)

    def zbody(j, _):
        def zrow(k, _2):
            a_v[j, pl.ds(k * 16, 16)] = zeros16
            return 0
        lax.fori_loop(0, _N // 16, zrow, 0)
        return 0

    lax.fori_loop(0, _RPW, zbody, 0)

    ones16 = jnp.ones((16,), jnp.float32)

    def ebody(i, _):
        s_v = src_v[pl.ds(i * 16, 16)]
        d_v = dst_v[pl.ds(i * 16, 16)]
        rel = d_v - base
        msk = jnp.logical_and(rel >= 0, rel < _RPW)
        row = jnp.where(msk, rel, 0)
        col = jnp.where(msk, s_v, 0)
        plsc.addupdate_scatter(a_v, [row, col], ones16, mask=msk)
        return 0

    lax.fori_loop(0, _E // 16, ebody, 0)
    pltpu.sync_copy(a_v, a_hbm.at[pl.ds(base, _RPW)])


def _lrelu(v):
    return jnp.where(v >= 0, v, NEG_SLOPE * v)


def _dot_hi(a, b):
    return jnp.dot(a, b, preferred_element_type=jnp.float32,
                   precision=lax.Precision.HIGHEST)


def _seg_dot(a16, v):
    f32 = jnp.float32
    hi = v.astype(jnp.bfloat16)
    r1 = v - hi.astype(f32)
    mid = r1.astype(jnp.bfloat16)
    lo = (r1 - mid.astype(f32)).astype(jnp.bfloat16)
    acc = jnp.dot(a16, hi, preferred_element_type=f32)
    acc = acc + jnp.dot(a16, mid, preferred_element_type=f32)
    acc = acc + jnp.dot(a16, lo, preferred_element_type=f32)
    return acc


def _stnet_body(x_ref, a_ref, Wg_ref, bg_ref, Wggc_ref, Wih_ref,
                Whh_ref, bih_ref, bhh_ref, W1_ref, b1_ref, W2_ref, b2_ref,
                s_ref, min_ref, loss_ref):
    f32 = jnp.float32
    n = x_ref.shape[0]
    num_l = Wggc_ref.shape[0]
    h_dim = Wg_ref.shape[1]

    iota_col = lax.broadcasted_iota(jnp.int32, (n, 1), 0)
    iota_row = lax.broadcasted_iota(jnp.int32, (1, n), 1)

    A = a_ref[...]

    rowsum = jnp.sum(A, axis=1, keepdims=True)
    deg = jnp.maximum(rowsum, 1.0)
    r = lax.rsqrt(deg)
    A16 = A.astype(jnp.bfloat16)

    xw = jnp.dot(x_ref[...], Wg_ref[...], preferred_element_type=f32)
    agg = r * _seg_dot(A16, r * xw) + bg_ref[...]
    x1 = _lrelu(agg)

    h = x1
    for i in range(num_l):
        hw = jnp.dot(h, Wggc_ref[i], preferred_element_type=f32)
        m = _seg_dot(A16, hw)
        gi = jnp.dot(m, Wih_ref[...], preferred_element_type=f32) + bih_ref[...]
        gh = jnp.dot(h, Whh_ref[...], preferred_element_type=f32) + bhh_ref[...]
        rg = jax.nn.sigmoid(gi[:, :h_dim] + gh[:, :h_dim])
        z = jax.nn.sigmoid(gi[:, h_dim:2 * h_dim] + gh[:, h_dim:2 * h_dim])
        nn_ = jnp.tanh(gi[:, 2 * h_dim:] + rg * gh[:, 2 * h_dim:])
        h = (1.0 - z) * nn_ + z * h

    x2 = _lrelu(h) + x1
    x3 = _lrelu(jnp.dot(x2, W1_ref[...], preferred_element_type=f32) + b1_ref[...])
    xf_col = jax.nn.sigmoid(_lrelu(jnp.dot(x3, W2_ref[...],
                                           preferred_element_type=f32) + b2_ref[...]))

    eye = (iota_col == iota_row).astype(f32)
    xf_row = lax.dot_general(xf_col, eye, (((0,), (0,)), ((), ())),
                             preferred_element_type=f32,
                             precision=lax.Precision.HIGHEST)

    gt = (xf_col > xf_row).astype(f32)
    tie = jnp.logical_and(xf_col == xf_row, iota_col < iota_row).astype(f32)
    rank_row = jnp.sum(gt + tie, axis=0, keepdims=True)

    icolf = iota_col.astype(f32)
    M = (rank_row <= icolf).astype(f32)
    colsum = lax.dot_general(A, jnp.ones((n, 1), f32), (((0,), (0,)), ((), ())),
                             preferred_element_type=f32)
    rc = rowsum + colsum
    t12 = jnp.dot(M, rc, preferred_element_type=f32)
    Bm = jnp.dot(M, A, preferred_element_type=f32)
    t3 = jnp.sum(Bm * M, axis=1, keepdims=True)
    cut = t12 - 2.0 * t3
    f_unreg = -cut
    f_sets = f_unreg + PENALTY * (icolf + 1.0)

    s_ref[...] = xf_col
    min_ref[...] = jnp.min(f_unreg, axis=0, keepdims=True)
    loss_ref[...] = jnp.sum(f_sets, axis=0, keepdims=True) / n


def kernel(x, edge_index, batch, W_gcn, b_gcn, W_ggc, W_ih, W_hh, b_ih, b_hh,
           W_lin1, b_lin1, W_lin2, b_lin2):
    n = x.shape[0]
    f32 = jnp.float32
    A = _build_adjacency(edge_index[0], edge_index[1])
    s, mn, ls = pl.pallas_call(
        _stnet_body,
        out_shape=(
            jax.ShapeDtypeStruct((n, 1), f32),
            jax.ShapeDtypeStruct((1, 1), f32),
            jax.ShapeDtypeStruct((1, 1), f32),
        ),
    )(x, A, W_gcn, b_gcn, W_ggc, W_ih, W_hh, b_ih, b_hh,
      W_lin1, b_lin1, W_lin2, b_lin2)
    return (s, mn.reshape(()), ls.reshape(()))
